# baseline (speedup 1.0000x reference)
import functools
import jax, jax.numpy as jnp
from jax.experimental import pallas as pl
from jax.experimental.pallas import tpu as pltpu
from jax.experimental.pallas import tpu_sc as plsc

def kernel(token_ids, word_lut, pe_table):
    seq_len, batch, _ = token_ids.shape
    vocab, dim = word_lut.shape
    n = seq_len * batch
    ids_flat = token_ids[:, :, 0].astype(jnp.int32).reshape(n)

    def stateful(refs):
        ids_ref, lut_ref, out_ref = refs
        mesh = plsc.ScalarSubcoreMesh(axis_name="core")
        ncores = mesh.num_cores
        rpc = n // ncores

        @pl.core_map(mesh=mesh)
        def _():
            core = jax.lax.axis_index("core")
            base = core * rpc

            def inner(ids_smem, sem):
                cp = pltpu.make_async_copy(
                    ids_ref.at[pl.ds(base, rpc)], ids_smem, sem)
                cp.start()
                cp.wait()

                def body(i, _):
                    tok = ids_smem[i]
                    pltpu.make_async_copy(
                        lut_ref.at[tok], out_ref.at[base + i], sem
                    ).start()
                    return ()
                jax.lax.fori_loop(0, rpc, body, ())
                pltpu.make_async_copy(
                    out_ref.at[pl.ds(base, rpc)],
                    out_ref.at[pl.ds(base, rpc)],
                    sem,
                ).wait()
            pl.run_scoped(
                inner,
                pltpu.SMEM((rpc,), jnp.int32),
                pltpu.SemaphoreType.DMA,
            )

    out = jnp.zeros((n, dim), jnp.float32)
    _, _, out = pl.run_state(stateful)((ids_flat, word_lut, out))
    return out.reshape(seq_len, batch, dim)


# speedup vs baseline: 12.1144x; 12.1144x over previous
import functools
import math

import jax
import jax.numpy as jnp
from jax.experimental import pallas as pl
from jax.experimental.pallas import tpu as pltpu


def _gather_embed_kernel(ids_ref, table_hbm, pe_ref, out_ref, gbuf, sem,
                         *, scale, tile_len, batch, n_inner):
    c = pl.program_id(0)
    j = pl.program_id(1)
    slot = jax.lax.rem(j, 2)
    rows = tile_len * batch

    def issue(tile_idx, dst_slot):
        base = tile_idx * rows
        for s in range(tile_len):
            for b in range(batch):
                tok = ids_ref[base + s * batch + b]
                pltpu.make_async_copy(
                    table_hbm.at[tok],
                    gbuf.at[dst_slot, s, b],
                    sem.at[dst_slot],
                ).start()

    @pl.when(j == 0)
    def _():
        issue(c * n_inner, slot)

    @pl.when(j + 1 < n_inner)
    def _():
        issue(c * n_inner + j + 1, 1 - slot)

    pltpu.make_async_copy(gbuf.at[slot], gbuf.at[slot], sem.at[slot]).wait()

    out_ref[...] = gbuf[slot] * scale + pe_ref[...]


def kernel(token_ids, word_lut, pe_table):
    seq_len, batch, nfeat = token_ids.shape
    assert nfeat == 1
    vocab, dim = word_lut.shape
    scale = float(math.sqrt(dim))

    tile_len = 32
    n_cores = 2
    n_inner = seq_len // tile_len // n_cores

    ids_flat = token_ids[:, :, 0].reshape(seq_len * batch).astype(jnp.int32)
    pe3 = pe_table[:seq_len].reshape(seq_len, 1, dim)

    body = functools.partial(
        _gather_embed_kernel,
        scale=scale, tile_len=tile_len, batch=batch, n_inner=n_inner,
    )

    grid_spec = pltpu.PrefetchScalarGridSpec(
        num_scalar_prefetch=1,
        grid=(n_cores, n_inner),
        in_specs=[
            pl.BlockSpec(memory_space=pl.ANY),
            pl.BlockSpec((tile_len, 1, dim),
                         lambda c, j, ids: (c * n_inner + j, 0, 0)),
        ],
        out_specs=pl.BlockSpec((tile_len, batch, dim),
                               lambda c, j, ids: (c * n_inner + j, 0, 0)),
        scratch_shapes=[
            pltpu.VMEM((2, tile_len, batch, dim), word_lut.dtype),
            pltpu.SemaphoreType.DMA((2,)),
        ],
    )

    out = pl.pallas_call(
        body,
        grid_spec=grid_spec,
        out_shape=jax.ShapeDtypeStruct((seq_len, batch, dim), word_lut.dtype),
        compiler_params=pltpu.CompilerParams(
            dimension_semantics=("arbitrary", "arbitrary"),
            disable_bounds_checks=True,
        ),
    )(ids_flat, word_lut, pe3)
    return out
